# baseline (speedup 1.0000x reference)
import jax
import jax.numpy as jnp
from jax.experimental import pallas as pl
from jax.experimental.pallas import tpu as pltpu

_ROWS = 1000000
_DIM = 64
_N = _ROWS * _DIM
_CBLK = 1_280_000
_CGRID = _N // _CBLK


def _copy_body(src_ref, dst_ref):
    dst_ref[...] = src_ref[...]


def _row_body(idx_ref, new_ref, tab_ref, out_ref):
    out_ref[...] = tab_ref[...]
    idx = idx_ref[0]
    local = idx - (idx // 8) * 8
    out_ref[pl.ds(local, 1), :] = new_ref[...]


def kernel(embeddings, emb_index, new_emb):
    idx = jnp.asarray(emb_index, jnp.int32).reshape(1)
    new_row = new_emb.reshape(1, _DIM)
    flat = embeddings.reshape(_N)
    out_flat = pl.pallas_call(
        _copy_body,
        grid=(_CGRID,),
        in_specs=[pl.BlockSpec((_CBLK,), lambda i: (i,))],
        out_specs=pl.BlockSpec((_CBLK,), lambda i: (i,)),
        out_shape=jax.ShapeDtypeStruct((_N,), embeddings.dtype),
    )(flat)
    table = out_flat.reshape(_ROWS, _DIM)
    grid_spec = pltpu.PrefetchScalarGridSpec(
        num_scalar_prefetch=1,
        grid=(1,),
        in_specs=[
            pl.BlockSpec((1, _DIM), lambda i, idx_ref: (0, 0)),
            pl.BlockSpec((8, _DIM), lambda i, idx_ref: (idx_ref[0] // 8, 0)),
        ],
        out_specs=pl.BlockSpec((8, _DIM), lambda i, idx_ref: (idx_ref[0] // 8, 0)),
    )
    return pl.pallas_call(
        _row_body,
        grid_spec=grid_spec,
        out_shape=jax.ShapeDtypeStruct((_ROWS, _DIM), embeddings.dtype),
        input_output_aliases={2: 0},
    )(idx, new_row, table)


# speedup vs baseline: 1.4955x; 1.4955x over previous
import jax
import jax.numpy as jnp
from jax.experimental import pallas as pl
from jax.experimental.pallas import tpu as pltpu

_ROWS = 1000000
_DIM = 64
_BLK = 8000
_NCH = _ROWS // _BLK
_NBUF = 8
_LOOK = 4


def _body(idx_ref, emb_hbm, new_ref, out_hbm, bufs, in_sems, out_sems):
    def in_copy(c):
        s = c % _NBUF
        return pltpu.make_async_copy(
            emb_hbm.at[pl.ds(c * _BLK, _BLK), :], bufs.at[s], in_sems.at[s]
        )

    def out_copy(c):
        s = c % _NBUF
        return pltpu.make_async_copy(
            bufs.at[s], out_hbm.at[pl.ds(c * _BLK, _BLK), :], out_sems.at[s]
        )

    for c in range(_LOOK):
        in_copy(c).start()
    for c in range(_NCH):
        nxt = c + _LOOK
        if nxt < _NCH:
            if nxt - _NBUF >= 0:
                out_copy(nxt - _NBUF).wait()
            in_copy(nxt).start()
        in_copy(c).wait()
        out_copy(c).start()
    for c in range(max(0, _NCH - _NBUF), _NCH):
        out_copy(c).wait()
    idx = idx_ref[0]
    rcp = pltpu.make_async_copy(new_ref, out_hbm.at[pl.ds(idx, 1), :], in_sems.at[0])
    rcp.start()
    rcp.wait()


def kernel(embeddings, emb_index, new_emb):
    idx = jnp.asarray(emb_index, jnp.int32).reshape(1)
    new_row = new_emb.reshape(1, _DIM)
    return pl.pallas_call(
        _body,
        in_specs=[
            pl.BlockSpec(memory_space=pltpu.SMEM),
            pl.BlockSpec(memory_space=pl.ANY),
            pl.BlockSpec(memory_space=pltpu.VMEM),
        ],
        out_specs=pl.BlockSpec(memory_space=pl.ANY),
        out_shape=jax.ShapeDtypeStruct((_ROWS, _DIM), embeddings.dtype),
        scratch_shapes=[
            pltpu.VMEM((_NBUF, _BLK, _DIM), jnp.float32),
            pltpu.SemaphoreType.DMA((_NBUF,)),
            pltpu.SemaphoreType.DMA((_NBUF,)),
        ],
    )(idx, embeddings, new_row)


# speedup vs baseline: 2.1955x; 1.4681x over previous
import jax
import jax.numpy as jnp
from jax.experimental import pallas as pl
from jax.experimental.pallas import tpu as pltpu

_ROWS = 1000000
_DIM = 64


def _row_body(idx_ref, new_ref, tab_ref, out_ref):
    out_ref[...] = tab_ref[...]
    idx = idx_ref[0]
    local = idx - (idx // 8) * 8
    out_ref[pl.ds(local, 1), :] = new_ref[...]


def kernel(embeddings, emb_index, new_emb):
    idx = jnp.asarray(emb_index, jnp.int32).reshape(1)
    new_row = new_emb.reshape(1, _DIM)
    grid_spec = pltpu.PrefetchScalarGridSpec(
        num_scalar_prefetch=1,
        grid=(1,),
        in_specs=[
            pl.BlockSpec((1, _DIM), lambda i, idx_ref: (0, 0)),
            pl.BlockSpec((8, _DIM), lambda i, idx_ref: (idx_ref[0] // 8, 0)),
        ],
        out_specs=pl.BlockSpec((8, _DIM), lambda i, idx_ref: (idx_ref[0] // 8, 0)),
    )
    return pl.pallas_call(
        _row_body,
        grid_spec=grid_spec,
        out_shape=jax.ShapeDtypeStruct((_ROWS, _DIM), embeddings.dtype),
        input_output_aliases={2: 0},
    )(idx, new_row, embeddings)
